# Initial kernel scaffold; baseline (speedup 1.0000x reference)
import jax
import jax.numpy as jnp
from jax.experimental import pallas as pl


def kernel(types_categories, times, types_specifics, types_specifics_pos_idx, cat_pos_idx, numerics, numerics_pos_idx, images, images_pos_idx, texts, texts_pos_idx, embed_weight, W_spec, b_spec, W_num, b_num, W_txt, b_txt, W_img, b_img, omega):
    raise NotImplementedError("write your pallas kernel here")


# speedup vs baseline: 1.0754x; 1.0754x over previous
import functools

import jax
import jax.numpy as jnp
from jax import lax
from jax.experimental import pallas as pl
from jax.experimental.pallas import tpu as pltpu
from jax.experimental.pallas import tpu_sc as plsc

B = 16
L = 2048
LP1 = L + 1
NCAT = 1024
NNUM = 1024
NIMG = 256
NTXT = 256
NSPEC = 1024
NTOT = NCAT + NNUM + NIMG + NTXT
EMB = 256
DSPEC = 64
DW = 128
DTXT = 768
DIMG = 768
NSCALES = 8

NC = 2
NS = 16
SLOTS = 4
NPASS = 2
BPC = B // NC
QTOK = NTOT // 4
QSPC = NSPEC // 4
LSTRIDE = 2056
ZR = 32
CS = 64
CG = 64
CE = 32


def _sc_gather_scatter(tspec, sidx, pos, tcat, times2, embw):
    mesh = plsc.VectorSubcoreMesh(
        core_axis_name="c", subcore_axis_name="s",
        num_cores=NC, num_subcores=NS)

    @functools.partial(
        pl.kernel,
        out_type=(
            jax.ShapeDtypeStruct((B, NTOT, EMB), jnp.float32),
            jax.ShapeDtypeStruct((B, NTOT, DW), jnp.float32),
            jax.ShapeDtypeStruct((B, NTOT), jnp.float32),
        ),
        mesh=mesh,
        compiler_params=pltpu.CompilerParams(needs_layout_passes=False),
        scratch_types=[
            pltpu.VMEM_SHARED((SLOTS * LSTRIDE, DW), jnp.float32),
            pltpu.VMEM((ZR, DW), jnp.float32),
            pltpu.VMEM((CS, DW), jnp.float32),
            pltpu.VMEM((QSPC,), jnp.int32),
            pltpu.VMEM((QSPC // CS, CS), jnp.int32),
            pltpu.VMEM((QTOK,), jnp.int32),
            pltpu.VMEM((QTOK // CG, CG), jnp.int32),
            pltpu.VMEM((QTOK // CE, CE), jnp.int32),
            pltpu.VMEM((L,), jnp.int32),
            pltpu.VMEM((L,), jnp.float32),
            pltpu.VMEM((QTOK,), jnp.float32),
            pltpu.VMEM((CG, DW), jnp.float32),
            pltpu.VMEM((CE, EMB), jnp.float32),
            pltpu.SemaphoreType.DMA,
        ],
    )
    def k(tspec_h, sidx_h, pos_h, tcat_h, times_h, embw_h,
          gcat_h, specg_h, timeg_h,
          tl, zbuf, srows, sidxl, sidx2, posv, gidx, cidx, tcv, tmv, tgv,
          sgbuf, ebuf, sem):
        c = lax.axis_index("c")
        s = lax.axis_index("s")
        slot = s // 4
        part = s % 4
        base_row = slot * LSTRIDE

        z16 = jnp.zeros((16,), jnp.float32)
        for i in range(ZR):
            for q in range(DW // 16):
                zbuf[i, pl.ds(q * 16, 16)] = z16

        for g in range(NPASS):
            b = c * BPC + g * SLOTS + slot
            if g > 0:
                plsc.subcore_barrier()

            row0 = base_row + part * (L // 4)
            for i in range((L // 4) // ZR):
                pltpu.sync_copy(zbuf, tl.at[pl.ds(row0 + i * ZR, ZR)])
            plsc.subcore_barrier()

            pltpu.sync_copy(sidx_h.at[b, pl.ds(part * QSPC, QSPC)], sidxl)
            for r in range(QSPC // CS):
                for q in range(CS // 16):
                    off = r * CS + q * 16
                    sidx2[r, pl.ds(q * 16, 16)] = (
                        sidxl[pl.ds(off, 16)] + base_row)
            for r in range(QSPC // CS):
                pltpu.sync_copy(
                    tspec_h.at[b, pl.ds(part * QSPC + r * CS, CS)], srows)
                pltpu.sync_copy(srows, tl.at[sidx2.at[r]], add=True)
            plsc.subcore_barrier()

            tok0 = part * QTOK
            pltpu.sync_copy(pos_h.at[b, pl.ds(tok0, QTOK)], posv)
            pltpu.sync_copy(tcat_h.at[b], tcv)
            pltpu.sync_copy(times_h.at[b], tmv)
            for r in range(QTOK // CG):
                for q in range(CG // 16):
                    off = r * CG + q * 16
                    p = posv[pl.ds(off, 16)]
                    cid = plsc.load_gather(tcv, [p]) + 1
                    cidx[off // CE, pl.ds(off % CE, 16)] = cid
                    gidx[r, pl.ds(q * 16, 16)] = p + base_row
                    tgv[pl.ds(off, 16)] = plsc.load_gather(tmv, [p])
            pltpu.sync_copy(tgv, timeg_h.at[b, pl.ds(tok0, QTOK)])
            for r in range(QTOK // CG):
                pltpu.async_copy(tl.at[gidx.at[r]], sgbuf, sem).wait()
                pltpu.sync_copy(
                    sgbuf, specg_h.at[b, pl.ds(tok0 + r * CG, CG)])
            for r in range(QTOK // CE):
                pltpu.async_copy(embw_h.at[cidx.at[r]], ebuf, sem).wait()
                pltpu.sync_copy(
                    ebuf, gcat_h.at[b, pl.ds(tok0 + r * CE, CE)])

    return k(tspec, sidx, pos, tcat, times2, embw)


CHT = 256
NJ = NTOT // CHT


def _tc_combine(g_cat, spec_g, numerics, images, texts,
                wts, wtn, wti, wtt, bs2, bn2, bi2, bt2, om2):
    def body(gc_ref, sg_ref, num_ref, img_ref, txt_ref,
             ws_ref, wn_ref, wi_ref, wt_ref,
             bs_ref, bn_ref, bi_ref, bt_ref, om_ref, out_ref):
        j = pl.program_id(1)
        acc = (gc_ref[0]
               + jnp.dot(sg_ref[0], ws_ref[...],
                         preferred_element_type=jnp.float32)
               + bs_ref[...])

        @pl.when(j < 4)
        def _():
            out_ref[0] = acc

        @pl.when(jnp.logical_and(j >= 4, j < 8))
        def _():
            xo = num_ref[0] * om_ref[...]
            enc = jnp.concatenate([jnp.sin(xo), jnp.cos(xo)], axis=-1)
            out_ref[0] = acc + jnp.dot(
                enc, wn_ref[...],
                preferred_element_type=jnp.float32) + bn_ref[...]

        @pl.when(j == 8)
        def _():
            out_ref[0] = acc + jnp.dot(
                img_ref[0], wi_ref[...],
                preferred_element_type=jnp.float32) + bi_ref[...]

        @pl.when(j == 9)
        def _():
            out_ref[0] = acc + jnp.dot(
                txt_ref[0], wt_ref[...],
                preferred_element_type=jnp.float32) + bt_ref[...]

    return pl.pallas_call(
        body,
        grid=(B, NJ),
        in_specs=[
            pl.BlockSpec((1, CHT, EMB), lambda b, j: (b, j, 0)),
            pl.BlockSpec((1, CHT, DW), lambda b, j: (b, j, 0)),
            pl.BlockSpec((1, CHT, 1), lambda b, j: (b, jnp.clip(j - 4, 0, 3), 0)),
            pl.BlockSpec((1, NIMG, DIMG), lambda b, j: (b, 0, 0)),
            pl.BlockSpec((1, NTXT, DTXT), lambda b, j: (b, 0, 0)),
            pl.BlockSpec((DW, EMB), lambda b, j: (0, 0)),
            pl.BlockSpec((2 * NSCALES, EMB), lambda b, j: (0, 0)),
            pl.BlockSpec((DIMG, EMB), lambda b, j: (0, 0)),
            pl.BlockSpec((DTXT, EMB), lambda b, j: (0, 0)),
            pl.BlockSpec((1, EMB), lambda b, j: (0, 0)),
            pl.BlockSpec((1, EMB), lambda b, j: (0, 0)),
            pl.BlockSpec((1, EMB), lambda b, j: (0, 0)),
            pl.BlockSpec((1, EMB), lambda b, j: (0, 0)),
            pl.BlockSpec((1, NSCALES), lambda b, j: (0, 0)),
        ],
        out_specs=pl.BlockSpec((1, CHT, EMB), lambda b, j: (b, j, 0)),
        out_shape=jax.ShapeDtypeStruct((B, NTOT, EMB), jnp.float32),
    )(g_cat, spec_g, numerics, images, texts,
      wts, wtn, wti, wtt, bs2, bn2, bi2, bt2, om2)


def kernel(types_categories, times, types_specifics, types_specifics_pos_idx,
           cat_pos_idx, numerics, numerics_pos_idx, images, images_pos_idx,
           texts, texts_pos_idx, embed_weight, W_spec, b_spec, W_num, b_num,
           W_txt, b_txt, W_img, b_img, omega):
    tcat = types_categories[..., 0]
    times2 = times[..., 0]
    pos_f = jnp.concatenate(
        [cat_pos_idx, numerics_pos_idx, images_pos_idx, texts_pos_idx],
        axis=1)
    pos_idx = pos_f.astype(jnp.int32)
    pos_res = pos_idx[..., 0] % L
    attn_mask = (pos_f >= 0).reshape(B, 1, 1, NTOT)
    sidx = (types_specifics_pos_idx.astype(jnp.int32) % LP1)[..., 0]
    tspec_pad = jnp.pad(types_specifics,
                        ((0, 0), (0, 0), (0, DW - DSPEC)))
    wts_pad = jnp.pad(W_spec.T, ((0, DW - DSPEC), (0, 0)))

    g_cat, spec_g, time_g = _sc_gather_scatter(
        tspec_pad, sidx, pos_res, tcat, times2, embed_weight)

    emb = _tc_combine(
        g_cat, spec_g, numerics, images, texts,
        wts_pad, W_num.T, W_img.T, W_txt.T,
        b_spec.reshape(1, EMB), b_num.reshape(1, EMB),
        b_img.reshape(1, EMB), b_txt.reshape(1, EMB),
        omega.reshape(1, NSCALES))

    return emb, pos_idx, attn_mask, time_g[..., None]


# speedup vs baseline: 1.1480x; 1.0676x over previous
import functools

import jax
import jax.numpy as jnp
from jax import lax
from jax.experimental import pallas as pl
from jax.experimental.pallas import tpu as pltpu
from jax.experimental.pallas import tpu_sc as plsc

B = 16
L = 2048
LP1 = L + 1
NCAT = 1024
NNUM = 1024
NIMG = 256
NTXT = 256
NSPEC = 1024
NTOT = NCAT + NNUM + NIMG + NTXT
EMB = 256
DSPEC = 64
DW = 128
DTXT = 768
DIMG = 768
NSCALES = 8

NC = 2
NS = 16
SLOTS = 4
NPASS = 2
BPC = B // NC
QTOK = NTOT // 4
QSPC = NSPEC // 4
LSTRIDE = 2056
ZR = 32
CS = 64
CG = 64
CE = 32


def _sc_gather_scatter(tspec, sidx, pos, tcat, times2, embw):
    mesh = plsc.VectorSubcoreMesh(
        core_axis_name="c", subcore_axis_name="s",
        num_cores=NC, num_subcores=NS)

    @functools.partial(
        pl.kernel,
        out_type=(
            jax.ShapeDtypeStruct((B, NTOT, EMB // 2), jnp.uint32),
            jax.ShapeDtypeStruct((B, NTOT, DW), jnp.float32),
            jax.ShapeDtypeStruct((B, NTOT), jnp.float32),
        ),
        mesh=mesh,
        compiler_params=pltpu.CompilerParams(needs_layout_passes=False),
        scratch_types=[
            pltpu.VMEM_SHARED((SLOTS * LSTRIDE, DW), jnp.float32),
            pltpu.VMEM((ZR, DW), jnp.float32),
            pltpu.VMEM((CS, DW), jnp.float32),
            pltpu.VMEM((QSPC,), jnp.int32),
            pltpu.VMEM((QSPC // CS, CS), jnp.int32),
            pltpu.VMEM((QTOK,), jnp.int32),
            pltpu.VMEM((QTOK // CG, CG), jnp.int32),
            pltpu.VMEM((QTOK // CE, CE), jnp.int32),
            pltpu.VMEM((L,), jnp.int32),
            pltpu.VMEM((L,), jnp.float32),
            pltpu.VMEM((QTOK,), jnp.float32),
            pltpu.VMEM((CG, DW), jnp.float32),
            pltpu.VMEM((CE, EMB // 2), jnp.uint32),
            pltpu.SemaphoreType.DMA,
        ],
    )
    def k(tspec_h, sidx_h, pos_h, tcat_h, times_h, embw_h,
          gcat_h, specg_h, timeg_h,
          tl, zbuf, srows, sidxl, sidx2, posv, gidx, cidx, tcv, tmv, tgv,
          sgbuf, ebuf, sem):
        c = lax.axis_index("c")
        s = lax.axis_index("s")
        slot = s // 4
        part = s % 4
        base_row = slot * LSTRIDE

        z16 = jnp.zeros((16,), jnp.float32)
        for i in range(ZR):
            for q in range(DW // 16):
                zbuf[i, pl.ds(q * 16, 16)] = z16

        for g in range(NPASS):
            b = c * BPC + g * SLOTS + slot
            if g > 0:
                plsc.subcore_barrier()

            row0 = base_row + part * (L // 4)
            for i in range((L // 4) // ZR):
                pltpu.sync_copy(zbuf, tl.at[pl.ds(row0 + i * ZR, ZR)])
            plsc.subcore_barrier()

            pltpu.sync_copy(sidx_h.at[b, pl.ds(part * QSPC, QSPC)], sidxl)
            for r in range(QSPC // CS):
                for q in range(CS // 16):
                    off = r * CS + q * 16
                    sidx2[r, pl.ds(q * 16, 16)] = (
                        sidxl[pl.ds(off, 16)] + base_row)
            for r in range(QSPC // CS):
                pltpu.sync_copy(
                    tspec_h.at[b, pl.ds(part * QSPC + r * CS, CS)], srows)
                pltpu.sync_copy(srows, tl.at[sidx2.at[r]], add=True)
            plsc.subcore_barrier()

            tok0 = part * QTOK
            pltpu.sync_copy(pos_h.at[b, pl.ds(tok0, QTOK)], posv)
            pltpu.sync_copy(tcat_h.at[b], tcv)
            pltpu.sync_copy(times_h.at[b], tmv)
            for r in range(QTOK // CG):
                for q in range(CG // 16):
                    off = r * CG + q * 16
                    p = posv[pl.ds(off, 16)]
                    cid = plsc.load_gather(tcv, [p]) + 1
                    cidx[off // CE, pl.ds(off % CE, 16)] = cid
                    gidx[r, pl.ds(q * 16, 16)] = p + base_row
                    tgv[pl.ds(off, 16)] = plsc.load_gather(tmv, [p])
            pltpu.sync_copy(tgv, timeg_h.at[b, pl.ds(tok0, QTOK)])
            for r in range(QTOK // CG):
                pltpu.async_copy(tl.at[gidx.at[r]], sgbuf, sem).wait()
                pltpu.sync_copy(
                    sgbuf, specg_h.at[b, pl.ds(tok0 + r * CG, CG)])
            for r in range(QTOK // CE):
                pltpu.async_copy(embw_h.at[cidx.at[r]], ebuf, sem).wait()
                pltpu.sync_copy(
                    ebuf, gcat_h.at[b, pl.ds(tok0 + r * CE, CE)])

    return k(tspec, sidx, pos, tcat, times2, embw)


CHT = 256
NJ = NTOT // CHT


def _tc_combine(g_cat, spec_g, numerics, images, texts,
                wts, wtn, wti, wtt, bs2, bn2, bi2, bt2, om2):
    def body(gc_ref, sg_ref, num_ref, img_ref, txt_ref,
             ws_ref, wn_ref, wi_ref, wt_ref,
             bs_ref, bn_ref, bi_ref, bt_ref, om_ref, out_ref):
        j = pl.program_id(1)
        gw = gc_ref[0]
        lo = lax.bitcast_convert_type(gw << 16, jnp.float32)
        hi = lax.bitcast_convert_type(gw & jnp.uint32(0xFFFF0000), jnp.float32)
        acc = (jnp.concatenate([lo, hi], axis=-1)
               + jnp.dot(sg_ref[0], ws_ref[...],
                         preferred_element_type=jnp.float32)
               + bs_ref[...])

        @pl.when(j < 4)
        def _():
            out_ref[0] = acc

        @pl.when(jnp.logical_and(j >= 4, j < 8))
        def _():
            xo = num_ref[0] * om_ref[...]
            enc = jnp.concatenate([jnp.sin(xo), jnp.cos(xo)], axis=-1)
            out_ref[0] = acc + jnp.dot(
                enc, wn_ref[...],
                preferred_element_type=jnp.float32) + bn_ref[...]

        @pl.when(j == 8)
        def _():
            out_ref[0] = acc + jnp.dot(
                img_ref[0], wi_ref[...],
                preferred_element_type=jnp.float32) + bi_ref[...]

        @pl.when(j == 9)
        def _():
            out_ref[0] = acc + jnp.dot(
                txt_ref[0], wt_ref[...],
                preferred_element_type=jnp.float32) + bt_ref[...]

    return pl.pallas_call(
        body,
        grid=(B, NJ),
        in_specs=[
            pl.BlockSpec((1, CHT, EMB // 2), lambda b, j: (b, j, 0)),
            pl.BlockSpec((1, CHT, DW), lambda b, j: (b, j, 0)),
            pl.BlockSpec((1, CHT, 1), lambda b, j: (b, jnp.clip(j - 4, 0, 3), 0)),
            pl.BlockSpec((1, NIMG, DIMG), lambda b, j: (b, 0, 0)),
            pl.BlockSpec((1, NTXT, DTXT), lambda b, j: (b, 0, 0)),
            pl.BlockSpec((DW, EMB), lambda b, j: (0, 0)),
            pl.BlockSpec((2 * NSCALES, EMB), lambda b, j: (0, 0)),
            pl.BlockSpec((DIMG, EMB), lambda b, j: (0, 0)),
            pl.BlockSpec((DTXT, EMB), lambda b, j: (0, 0)),
            pl.BlockSpec((1, EMB), lambda b, j: (0, 0)),
            pl.BlockSpec((1, EMB), lambda b, j: (0, 0)),
            pl.BlockSpec((1, EMB), lambda b, j: (0, 0)),
            pl.BlockSpec((1, EMB), lambda b, j: (0, 0)),
            pl.BlockSpec((1, NSCALES), lambda b, j: (0, 0)),
        ],
        out_specs=pl.BlockSpec((1, CHT, EMB), lambda b, j: (b, j, 0)),
        out_shape=jax.ShapeDtypeStruct((B, NTOT, EMB), jnp.float32),
    )(g_cat, spec_g, numerics, images, texts,
      wts, wtn, wti, wtt, bs2, bn2, bi2, bt2, om2)


def kernel(types_categories, times, types_specifics, types_specifics_pos_idx,
           cat_pos_idx, numerics, numerics_pos_idx, images, images_pos_idx,
           texts, texts_pos_idx, embed_weight, W_spec, b_spec, W_num, b_num,
           W_txt, b_txt, W_img, b_img, omega):
    tcat = types_categories[..., 0]
    times2 = times[..., 0]
    pos_f = jnp.concatenate(
        [cat_pos_idx, numerics_pos_idx, images_pos_idx, texts_pos_idx],
        axis=1)
    pos_idx = pos_f.astype(jnp.int32)
    pos_res = pos_idx[..., 0] % L
    attn_mask = (pos_f >= 0).reshape(B, 1, 1, NTOT)
    sidx = (types_specifics_pos_idx.astype(jnp.int32) % LP1)[..., 0]
    tspec_pad = jnp.pad(types_specifics,
                        ((0, 0), (0, 0), (0, DW - DSPEC)))
    wts_pad = jnp.pad(W_spec.T, ((0, DW - DSPEC), (0, 0)))

    ew_bf = embed_weight.astype(jnp.bfloat16)
    ew_lo = lax.bitcast_convert_type(
        ew_bf[:, :EMB // 2], jnp.uint16).astype(jnp.uint32)
    ew_hi = lax.bitcast_convert_type(
        ew_bf[:, EMB // 2:], jnp.uint16).astype(jnp.uint32)
    ew_packed = ew_lo | (ew_hi << 16)

    g_cat, spec_g, time_g = _sc_gather_scatter(
        tspec_pad, sidx, pos_res, tcat, times2, ew_packed)

    emb = _tc_combine(
        g_cat, spec_g, numerics, images, texts,
        wts_pad, W_num.T, W_img.T, W_txt.T,
        b_spec.reshape(1, EMB), b_num.reshape(1, EMB),
        b_img.reshape(1, EMB), b_txt.reshape(1, EMB),
        omega.reshape(1, NSCALES))

    return emb, pos_idx, attn_mask, time_g[..., None]


# speedup vs baseline: 1.3654x; 1.1893x over previous
import functools

import jax
import jax.numpy as jnp
from jax import lax
from jax.experimental import pallas as pl
from jax.experimental.pallas import tpu as pltpu
from jax.experimental.pallas import tpu_sc as plsc

B = 16
L = 2048
LP1 = L + 1
NCAT = 1024
NNUM = 1024
NIMG = 256
NTXT = 256
NSPEC = 1024
NTOT = NCAT + NNUM + NIMG + NTXT
EMB = 256
DSPEC = 64
DW = 128
DTXT = 768
DIMG = 768
NSCALES = 8

NC = 2
NS = 16
SLOTS = 4
NPASS = 2
BPC = B // NC
QTOK = NTOT // 4
QSPC = NSPEC // 4
LSTRIDE = 2056
ZR = 32
CG = 64
NG = QTOK // CG
CS = 128
NS_CH = QSPC // CS


def _sc_gather_scatter(tspec, sidx, pos, tcat, times2, embw):
    mesh = plsc.VectorSubcoreMesh(
        core_axis_name="c", subcore_axis_name="s",
        num_cores=NC, num_subcores=NS)

    @functools.partial(
        pl.kernel,
        out_type=(
            jax.ShapeDtypeStruct((B, NTOT, EMB // 2), jnp.uint32),
            jax.ShapeDtypeStruct((B, NTOT, DW), jnp.float32),
            jax.ShapeDtypeStruct((B, NTOT), jnp.float32),
        ),
        mesh=mesh,
        compiler_params=pltpu.CompilerParams(needs_layout_passes=False),
        scratch_types=[
            pltpu.VMEM_SHARED((SLOTS * LSTRIDE, DW), jnp.float32),
            pltpu.VMEM((ZR, DW), jnp.float32),
            pltpu.VMEM((CS, DW), jnp.float32),
            pltpu.VMEM((QSPC,), jnp.int32),
            pltpu.VMEM((NS_CH, CS), jnp.int32),
            pltpu.VMEM((QTOK,), jnp.int32),
            pltpu.VMEM((NG, CG), jnp.int32),
            pltpu.VMEM((NG, CG), jnp.int32),
            pltpu.VMEM((L,), jnp.int32),
            pltpu.VMEM((L,), jnp.float32),
            pltpu.VMEM((QTOK,), jnp.float32),
            pltpu.VMEM((2, CG, DW), jnp.float32),
            pltpu.VMEM((2, CG, EMB // 2), jnp.uint32),
        ] + [pltpu.SemaphoreType.DMA] * 14,
    )
    def k(tspec_h, sidx_h, pos_h, tcat_h, times_h, embw_h,
          gcat_h, specg_h, timeg_h,
          tl, zbuf, srows, sidxl, sidx2, posv, gidx, cidx, tcv, tmv, tgv,
          sgb, ebf,
          zsem, psem, csemt, msem, lsem, isem, tssem,
          ge0, ge1, oe0, oe1, gt0, gt1, st0):
        c = lax.axis_index("c")
        s = lax.axis_index("s")
        slot = s // 4
        part = s % 4
        base_row = slot * LSTRIDE
        gesem = [ge0, ge1]
        oesem = [oe0, oe1]
        gtsem = [gt0, gt1]
        stsem = [st0, tssem]

        z16 = jnp.zeros((16,), jnp.float32)
        for i in range(ZR):
            for q in range(DW // 16):
                zbuf[i, pl.ds(q * 16, 16)] = z16

        for g in range(NPASS):
            b = c * BPC + g * SLOTS + slot
            tok0 = part * QTOK
            if g > 0:
                plsc.subcore_barrier()

            row0 = base_row + part * (L // 4)
            zh = [pltpu.async_copy(
                      zbuf, tl.at[pl.ds(row0 + i * ZR, ZR)], zsem)
                  for i in range((L // 4) // ZR)]

            hp = pltpu.async_copy(pos_h.at[b, pl.ds(tok0, QTOK)], posv, psem)
            ht = pltpu.async_copy(tcat_h.at[b], tcv, csemt)
            hm = pltpu.async_copy(times_h.at[b], tmv, msem)
            hs = pltpu.async_copy(
                tspec_h.at[b, pl.ds(part * QSPC, CS)], srows, lsem)
            hi = pltpu.async_copy(
                sidx_h.at[b, pl.ds(part * QSPC, QSPC)], sidxl, isem)

            hp.wait(); ht.wait(); hm.wait()
            for t in range(QTOK // 16):
                off = t * 16
                p = posv[pl.ds(off, 16)]
                cid = plsc.load_gather(tcv, [p]) + 1
                cidx[t // 4, pl.ds((t % 4) * 16, 16)] = cid
                gidx[t // 4, pl.ds((t % 4) * 16, 16)] = p + base_row
                tgv[pl.ds(off, 16)] = plsc.load_gather(tmv, [p])
            hts = pltpu.async_copy(
                tgv, timeg_h.at[b, pl.ds(tok0, QTOK)], tssem)

            eh = [None] * NG
            oh = [None] * NG
            eh[0] = pltpu.async_copy(embw_h.at[cidx.at[0]], ebf.at[0], ge0)
            eh[1] = pltpu.async_copy(embw_h.at[cidx.at[1]], ebf.at[1], ge1)

            hi.wait()
            for t in range(QSPC // 16):
                sidx2[t // 8, pl.ds((t % 8) * 16, 16)] = (
                    sidxl[pl.ds(t * 16, 16)] + base_row)
            for h in zh:
                h.wait()
            plsc.subcore_barrier()
            eh[0].wait()
            oh[0] = pltpu.async_copy(
                ebf.at[0], gcat_h.at[b, pl.ds(tok0, CG)], oe0)
            for r in range(NS_CH):
                if r > 0:
                    hs = pltpu.async_copy(
                        tspec_h.at[b, pl.ds(part * QSPC + r * CS, CS)],
                        srows, lsem)
                hs.wait()
                pltpu.sync_copy(srows, tl.at[sidx2.at[r]], add=True)
            hts.wait()
            plsc.subcore_barrier()

            gh = [None] * NG
            sth = [None] * NG
            for r in range(NG + 1):
                if r < NG:
                    if r >= 2:
                        oh[r - 2].wait()
                        eh[r] = pltpu.async_copy(
                            embw_h.at[cidx.at[r]], ebf.at[r % 2],
                            gesem[r % 2])
                        sth[r - 2].wait()
                    gh[r] = pltpu.async_copy(
                        tl.at[gidx.at[r]], sgb.at[r % 2], gtsem[r % 2])
                if r >= 1:
                    rr = r - 1
                    if rr >= 1:
                        eh[rr].wait()
                        oh[rr] = pltpu.async_copy(
                            ebf.at[rr % 2],
                            gcat_h.at[b, pl.ds(tok0 + rr * CG, CG)],
                            oesem[rr % 2])
                    gh[rr].wait()
                    sth[rr] = pltpu.async_copy(
                        sgb.at[rr % 2],
                        specg_h.at[b, pl.ds(tok0 + rr * CG, CG)],
                        stsem[rr % 2])
            for h in (oh[NG - 2], oh[NG - 1], sth[NG - 2], sth[NG - 1]):
                h.wait()

    return k(tspec, sidx, pos, tcat, times2, embw)


CHT = 256
NJ = NTOT // CHT


def _tc_combine(g_cat, spec_g, numerics, images, texts,
                wts, wtn, wti, wtt, bs2, bn2, bi2, bt2, om2):
    def body(gc_ref, sg_ref, num_ref, img_ref, txt_ref,
             ws_ref, wn_ref, wi_ref, wt_ref,
             bs_ref, bn_ref, bi_ref, bt_ref, om_ref, out_ref):
        j = pl.program_id(1)
        gw = gc_ref[0]
        lo = lax.bitcast_convert_type(gw << 16, jnp.float32)
        hi = lax.bitcast_convert_type(gw & jnp.uint32(0xFFFF0000), jnp.float32)
        acc = (jnp.concatenate([lo, hi], axis=-1)
               + jnp.dot(sg_ref[0], ws_ref[...],
                         preferred_element_type=jnp.float32)
               + bs_ref[...])

        @pl.when(j < 4)
        def _():
            out_ref[0] = acc

        @pl.when(jnp.logical_and(j >= 4, j < 8))
        def _():
            xo = num_ref[0] * om_ref[...]
            enc = jnp.concatenate([jnp.sin(xo), jnp.cos(xo)], axis=-1)
            out_ref[0] = acc + jnp.dot(
                enc, wn_ref[...],
                preferred_element_type=jnp.float32) + bn_ref[...]

        @pl.when(j == 8)
        def _():
            out_ref[0] = acc + jnp.dot(
                img_ref[0], wi_ref[...],
                preferred_element_type=jnp.float32) + bi_ref[...]

        @pl.when(j == 9)
        def _():
            out_ref[0] = acc + jnp.dot(
                txt_ref[0], wt_ref[...],
                preferred_element_type=jnp.float32) + bt_ref[...]

    return pl.pallas_call(
        body,
        grid=(B, NJ),
        in_specs=[
            pl.BlockSpec((1, CHT, EMB // 2), lambda b, j: (b, j, 0)),
            pl.BlockSpec((1, CHT, DW), lambda b, j: (b, j, 0)),
            pl.BlockSpec((1, CHT, 1), lambda b, j: (b, jnp.clip(j - 4, 0, 3), 0)),
            pl.BlockSpec((1, NIMG, DIMG), lambda b, j: (b, 0, 0)),
            pl.BlockSpec((1, NTXT, DTXT), lambda b, j: (b, 0, 0)),
            pl.BlockSpec((DW, EMB), lambda b, j: (0, 0)),
            pl.BlockSpec((2 * NSCALES, EMB), lambda b, j: (0, 0)),
            pl.BlockSpec((DIMG, EMB), lambda b, j: (0, 0)),
            pl.BlockSpec((DTXT, EMB), lambda b, j: (0, 0)),
            pl.BlockSpec((1, EMB), lambda b, j: (0, 0)),
            pl.BlockSpec((1, EMB), lambda b, j: (0, 0)),
            pl.BlockSpec((1, EMB), lambda b, j: (0, 0)),
            pl.BlockSpec((1, EMB), lambda b, j: (0, 0)),
            pl.BlockSpec((1, NSCALES), lambda b, j: (0, 0)),
        ],
        out_specs=pl.BlockSpec((1, CHT, EMB), lambda b, j: (b, j, 0)),
        out_shape=jax.ShapeDtypeStruct((B, NTOT, EMB), jnp.float32),
    )(g_cat, spec_g, numerics, images, texts,
      wts, wtn, wti, wtt, bs2, bn2, bi2, bt2, om2)


def kernel(types_categories, times, types_specifics, types_specifics_pos_idx,
           cat_pos_idx, numerics, numerics_pos_idx, images, images_pos_idx,
           texts, texts_pos_idx, embed_weight, W_spec, b_spec, W_num, b_num,
           W_txt, b_txt, W_img, b_img, omega):
    tcat = types_categories[..., 0]
    times2 = times[..., 0]
    pos_f = jnp.concatenate(
        [cat_pos_idx, numerics_pos_idx, images_pos_idx, texts_pos_idx],
        axis=1)
    pos_idx = pos_f.astype(jnp.int32)
    pos_res = pos_idx[..., 0] % L
    attn_mask = (pos_f >= 0).reshape(B, 1, 1, NTOT)
    sidx = (types_specifics_pos_idx.astype(jnp.int32) % LP1)[..., 0]
    tspec_pad = jnp.pad(types_specifics,
                        ((0, 0), (0, 0), (0, DW - DSPEC)))
    wts_pad = jnp.pad(W_spec.T, ((0, DW - DSPEC), (0, 0)))

    ew_bf = embed_weight.astype(jnp.bfloat16)
    ew_lo = lax.bitcast_convert_type(
        ew_bf[:, :EMB // 2], jnp.uint16).astype(jnp.uint32)
    ew_hi = lax.bitcast_convert_type(
        ew_bf[:, EMB // 2:], jnp.uint16).astype(jnp.uint32)
    ew_packed = ew_lo | (ew_hi << 16)

    g_cat, spec_g, time_g = _sc_gather_scatter(
        tspec_pad, sidx, pos_res, tcat, times2, ew_packed)

    emb = _tc_combine(
        g_cat, spec_g, numerics, images, texts,
        wts_pad, W_num.T, W_img.T, W_txt.T,
        b_spec.reshape(1, EMB), b_num.reshape(1, EMB),
        b_img.reshape(1, EMB), b_txt.reshape(1, EMB),
        omega.reshape(1, NSCALES))

    return emb, pos_idx, attn_mask, time_g[..., None]
